# baseline (speedup 1.0000x reference)
import functools

import jax
import jax.numpy as jnp
from jax import lax
from jax.experimental import pallas as pl
from jax.experimental.pallas import tpu as pltpu
from jax.experimental.pallas import tpu_sc as plsc

N = 10000
D = 128
E = 320000
NC = 2
NS = 16
TILES = NC * NS
KD = 128
CHD = 80
E_PAD = TILES * KD * CHD
K = 128
CH = CHD
PHASES = ((0, 40), (40, 40))
PBUF = 40
N2 = 10112
DUMMY = N
RPT = N2 // NS

_mesh = plsc.VectorSubcoreMesh(core_axis_name="c", subcore_axis_name="s",
                               num_cores=NC, num_subcores=NS)


@functools.partial(
    pl.kernel,
    out_type=jax.ShapeDtypeStruct((NC * N2,), jnp.float32),
    mesh=_mesh,
    scratch_types=[
        pltpu.VMEM((CHD, KD), jnp.int32),
        pltpu.VMEM((KD,), jnp.float32),
        pltpu.VMEM((RPT,), jnp.float32),
        pltpu.VMEM_SHARED((N2,), jnp.float32),
    ],
)
def _deg_kernel(dst_hbm, ones_hbm, zcol_hbm, out_hbm, dst_v, ones_v, zbuf,
                deg_sh):
    c = lax.axis_index("c")
    s = lax.axis_index("s")
    pltpu.sync_copy(dst_hbm.at[c * NS + s], dst_v)
    pltpu.sync_copy(ones_hbm, ones_v)
    pltpu.sync_copy(zcol_hbm.at[pl.ds(s * RPT, RPT)], zbuf)
    pltpu.sync_copy(zbuf, deg_sh.at[pl.ds(s * RPT, RPT)])
    plsc.subcore_barrier()

    def body(j, carry):
        pltpu.sync_copy(ones_v, deg_sh.at[dst_v.at[j]], add=True)
        return carry

    lax.fori_loop(0, CHD, body, 0)
    plsc.subcore_barrier()
    pltpu.sync_copy(deg_sh.at[pl.ds(s * RPT, RPT)], zbuf)
    pltpu.sync_copy(zbuf, out_hbm.at[pl.ds(c * N2 + s * RPT, RPT)])


@functools.partial(
    pl.kernel,
    out_type=(jax.ShapeDtypeStruct((NC, N2, D), jnp.float32),
              jax.ShapeDtypeStruct((NC * N2,), jnp.float32)),
    mesh=_mesh,
    scratch_types=[
        pltpu.VMEM((PBUF, K), jnp.int32),
        pltpu.VMEM((PBUF, K), jnp.int32),
        pltpu.VMEM((PBUF, K), jnp.float32),
        pltpu.VMEM((2, K, D), jnp.float32),
        pltpu.VMEM_SHARED((N2, D), jnp.float32),
        pltpu.VMEM_SHARED((N2,), jnp.float32),
        pltpu.SemaphoreType.DMA((2,)),
        pltpu.SemaphoreType.DMA((2,)),
        pltpu.SemaphoreType.DMA,
        pltpu.SemaphoreType.DMA,
    ],
)
def _agg_kernel(y_hbm, r_hbm, src_hbm, dst_hbm, zrows_hbm, zcol_hbm,
                acc_hbm, t_hbm,
                src_v, dst_v, rvals_v, rows_v, acc_sh, t_sh, gy, sy, gt, st):
    c = lax.axis_index("c")
    s = lax.axis_index("s")
    wid = c * NS + s

    pltpu.sync_copy(zrows_hbm.at[pl.ds(s * RPT, K)], rows_v.at[0])

    def zb(i, carry):
        pltpu.sync_copy(rows_v.at[0], acc_sh.at[pl.ds(s * RPT + i * K, K)])
        return carry

    lax.fori_loop(0, RPT // K, zb, 0)
    pltpu.sync_copy(rows_v.at[0, pl.ds(0, RPT % K)],
                    acc_sh.at[pl.ds(s * RPT + (RPT // K) * K, RPT % K)])
    pltpu.sync_copy(zcol_hbm.at[pl.ds(s * RPT, K)], rvals_v.at[0])

    def zt(i, carry):
        pltpu.sync_copy(rvals_v.at[0], t_sh.at[pl.ds(s * RPT + i * K, K)])
        return carry

    lax.fori_loop(0, RPT // K, zt, 0)
    pltpu.sync_copy(rvals_v.at[0, pl.ds(0, RPT % K)],
                    t_sh.at[pl.ds(s * RPT + (RPT // K) * K, RPT % K)])
    plsc.subcore_barrier()

    for pstart, plen in PHASES:
        pltpu.sync_copy(src_hbm.at[wid, pl.ds(pstart, plen)],
                        src_v.at[pl.ds(0, plen)])
        pltpu.sync_copy(dst_hbm.at[wid, pl.ds(pstart, plen)],
                        dst_v.at[pl.ds(0, plen)])

        for b in range(2):
            pltpu.async_copy(y_hbm.at[src_v.at[b]], rows_v.at[b], gy.at[b])

        def tg(j, carry):
            pltpu.async_copy(r_hbm.at[dst_v.at[j]], rvals_v.at[j], gt)
            return carry

        lax.fori_loop(0, plen, tg, 0)

        def pair(i, carry):
            j0 = 2 * i
            j1 = 2 * i + 1
            pltpu.make_async_copy(y_hbm.at[src_v.at[j0]], rows_v.at[0],
                                  gy.at[0]).wait()
            pltpu.async_copy(rows_v.at[0], acc_sh.at[dst_v.at[j0]],
                             sy.at[0], add=True)
            pltpu.make_async_copy(y_hbm.at[src_v.at[j1]], rows_v.at[1],
                                  gy.at[1]).wait()
            pltpu.async_copy(rows_v.at[1], acc_sh.at[dst_v.at[j1]],
                             sy.at[1], add=True)
            pltpu.make_async_copy(rows_v.at[0], acc_sh.at[dst_v.at[j0]],
                                  sy.at[0]).wait()

            @pl.when(j0 + 2 < plen)
            def _():
                pltpu.async_copy(y_hbm.at[src_v.at[j0 + 2]], rows_v.at[0],
                                 gy.at[0])

            pltpu.make_async_copy(rows_v.at[1], acc_sh.at[dst_v.at[j1]],
                                  sy.at[1]).wait()

            @pl.when(j1 + 2 < plen)
            def _():
                pltpu.async_copy(y_hbm.at[src_v.at[j1 + 2]], rows_v.at[1],
                                 gy.at[1])
            return carry

        lax.fori_loop(0, plen // 2, pair, 0)

        def ts(j, carry):
            pltpu.make_async_copy(r_hbm.at[dst_v.at[j]], rvals_v.at[j],
                                  gt).wait()
            pltpu.async_copy(rvals_v.at[j], t_sh.at[src_v.at[j]], st,
                             add=True)
            return carry

        lax.fori_loop(0, plen, ts, 0)

        def tsw(j, carry):
            pltpu.make_async_copy(rvals_v.at[j], t_sh.at[src_v.at[j]],
                                  st).wait()
            return carry

        lax.fori_loop(0, plen, tsw, 0)

    plsc.subcore_barrier()

    def cpout(i, carry):
        pltpu.sync_copy(acc_sh.at[pl.ds(s * RPT + i * K, K)], rows_v.at[0])
        pltpu.sync_copy(rows_v.at[0], acc_hbm.at[c, pl.ds(s * RPT + i * K, K)])
        return carry

    lax.fori_loop(0, RPT // K, cpout, 0)
    pltpu.sync_copy(acc_sh.at[pl.ds(s * RPT + (RPT // K) * K, RPT % K)],
                    rows_v.at[0, pl.ds(0, RPT % K)])
    pltpu.sync_copy(rows_v.at[0, pl.ds(0, RPT % K)],
                    acc_hbm.at[c, pl.ds(s * RPT + (RPT // K) * K, RPT % K)])

    def tout(i, carry):
        pltpu.sync_copy(t_sh.at[pl.ds(s * RPT + i * K, K)], rvals_v.at[0])
        pltpu.sync_copy(rvals_v.at[0],
                        t_hbm.at[pl.ds(c * N2 + s * RPT + i * K, K)])
        return carry

    lax.fori_loop(0, RPT // K, tout, 0)
    pltpu.sync_copy(t_sh.at[pl.ds(s * RPT + (RPT // K) * K, RPT % K)],
                    rvals_v.at[0, pl.ds(0, RPT % K)])
    pltpu.sync_copy(rvals_v.at[0, pl.ds(0, RPT % K)],
                    t_hbm.at[pl.ds(c * N2 + s * RPT + (RPT // K) * K,
                                   RPT % K)])


def _scale_body(xp_ref, w1_ref, degp_ref, y_ref, r_ref):
    deg = jnp.sum(degp_ref[...], axis=1, keepdims=True) + 1.0
    r = lax.rsqrt(deg)
    xw = jnp.dot(xp_ref[...], w1_ref[...],
                 preferred_element_type=jnp.float32)
    y_ref[...] = r * xw
    r_ref[...] = r


def _final_body(acc_ref, y_ref, tp_ref, r_ref, b1_ref, w2_ref, b2_ref, o_ref):
    r = r_ref[...]
    a = acc_ref[0] + acc_ref[1] + y_ref[...]
    h1 = jnp.maximum(r * a + b1_ref[...], 0.0)
    t = jnp.sum(tp_ref[...], axis=1, keepdims=True) + r
    valid = lax.broadcasted_iota(jnp.int32, (N2, 1), 0) < N
    cvec = jnp.where(valid, r * t, 0.0)
    u = jnp.sum(cvec * h1, axis=0, keepdims=True)
    o_ref[...] = (u * (1.0 / N)) @ w2_ref[...] + b2_ref[...]


def kernel(x, edge_index, W1, b1, W2, b2):
    ei = edge_index.astype(jnp.int32)
    padv = DUMMY + jnp.arange(E_PAD - E, dtype=jnp.int32) % (N2 - N)
    src_flat = jnp.concatenate([ei[0], padv])
    dst_flat = jnp.concatenate([ei[1], padv])
    xp = jnp.zeros((N2, D), jnp.float32).at[:N].set(x)
    ones = jnp.ones((KD,), jnp.float32)
    zcol = jnp.zeros((N2,), jnp.float32)
    zrows = jnp.zeros((N2, D), jnp.float32)

    deg_part = _deg_kernel(dst_flat.reshape(TILES, CHD, KD), ones,
                           zcol).reshape(NC, N2)

    y, r_col = pl.pallas_call(
        _scale_body,
        out_shape=[jax.ShapeDtypeStruct((N2, D), jnp.float32),
                   jax.ShapeDtypeStruct((N2, 1), jnp.float32)],
    )(xp, W1, deg_part.T)

    acc, t_part = _agg_kernel(y, r_col.reshape(N2),
                              src_flat.reshape(TILES, CH, K),
                              dst_flat.reshape(TILES, CH, K), zrows, zcol)

    out = pl.pallas_call(
        _final_body,
        out_shape=jax.ShapeDtypeStruct((1, D), jnp.float32),
    )(acc, y, t_part.reshape(NC, N2).T, r_col, b1.reshape(1, D), W2,
      b2.reshape(1, D))
    return out


# speedup vs baseline: 1.2913x; 1.2913x over previous
import functools

import jax
import jax.numpy as jnp
from jax import lax
from jax.experimental import pallas as pl
from jax.experimental.pallas import tpu as pltpu
from jax.experimental.pallas import tpu_sc as plsc

N = 10000
D = 128
E = 320000
NC = 2
NS = 16
TILES = NC * NS
KD = 128
CHD = 80
E_PAD = TILES * KD * CHD
K = 128
CH = CHD
PHASES = ((0, 40), (40, 40))
PBUF = 40
N2 = 10112
DUMMY = N
RPT = N2 // NS

_mesh = plsc.VectorSubcoreMesh(core_axis_name="c", subcore_axis_name="s",
                               num_cores=NC, num_subcores=NS)


@functools.partial(
    pl.kernel,
    out_type=jax.ShapeDtypeStruct((NC * N2,), jnp.float32),
    mesh=_mesh,
    scratch_types=[
        pltpu.VMEM((CHD, KD), jnp.int32),
        pltpu.VMEM((KD,), jnp.float32),
        pltpu.VMEM((RPT,), jnp.float32),
        pltpu.VMEM_SHARED((N2,), jnp.float32),
    ],
)
def _deg_kernel(dst_hbm, ones_hbm, zcol_hbm, out_hbm, dst_v, ones_v, zbuf,
                deg_sh):
    c = lax.axis_index("c")
    s = lax.axis_index("s")
    pltpu.sync_copy(dst_hbm.at[c * NS + s], dst_v)
    pltpu.sync_copy(ones_hbm, ones_v)
    pltpu.sync_copy(zcol_hbm.at[pl.ds(s * RPT, RPT)], zbuf)
    pltpu.sync_copy(zbuf, deg_sh.at[pl.ds(s * RPT, RPT)])
    plsc.subcore_barrier()

    def body(j, carry):
        pltpu.sync_copy(ones_v, deg_sh.at[dst_v.at[j]], add=True)
        return carry

    lax.fori_loop(0, CHD, body, 0)
    plsc.subcore_barrier()
    pltpu.sync_copy(deg_sh.at[pl.ds(s * RPT, RPT)], zbuf)
    pltpu.sync_copy(zbuf, out_hbm.at[pl.ds(c * N2 + s * RPT, RPT)])


@functools.partial(
    pl.kernel,
    out_type=(jax.ShapeDtypeStruct((NC, N2, D), jnp.float32),
              jax.ShapeDtypeStruct((NC * N2,), jnp.float32)),
    mesh=_mesh,
    scratch_types=[
        pltpu.VMEM((PBUF, K), jnp.int32),
        pltpu.VMEM((PBUF, K), jnp.int32),
        pltpu.VMEM((PBUF, K), jnp.float32),
        pltpu.VMEM((2, K, D), jnp.float32),
        pltpu.VMEM_SHARED((N2, D), jnp.float32),
        pltpu.VMEM_SHARED((N2,), jnp.float32),
        pltpu.SemaphoreType.DMA((2,)),
        pltpu.SemaphoreType.DMA((2,)),
        pltpu.SemaphoreType.DMA,
        pltpu.SemaphoreType.DMA,
    ],
)
def _agg_kernel(y_hbm, r_hbm, src_hbm, dst_hbm, zrows_hbm, zcol_hbm,
                acc_hbm, t_hbm,
                src_v, dst_v, rvals_v, rows_v, acc_sh, t_sh, gy, sy, gt, st):
    c = lax.axis_index("c")
    s = lax.axis_index("s")
    wid = c * NS + s

    pltpu.sync_copy(zrows_hbm.at[pl.ds(s * RPT, K)], rows_v.at[0])

    def zb(i, carry):
        pltpu.sync_copy(rows_v.at[0], acc_sh.at[pl.ds(s * RPT + i * K, K)])
        return carry

    lax.fori_loop(0, RPT // K, zb, 0)
    pltpu.sync_copy(rows_v.at[0, pl.ds(0, RPT % K)],
                    acc_sh.at[pl.ds(s * RPT + (RPT // K) * K, RPT % K)])
    pltpu.sync_copy(zcol_hbm.at[pl.ds(s * RPT, K)], rvals_v.at[0])

    def zt(i, carry):
        pltpu.sync_copy(rvals_v.at[0], t_sh.at[pl.ds(s * RPT + i * K, K)])
        return carry

    lax.fori_loop(0, RPT // K, zt, 0)
    pltpu.sync_copy(rvals_v.at[0, pl.ds(0, RPT % K)],
                    t_sh.at[pl.ds(s * RPT + (RPT // K) * K, RPT % K)])
    plsc.subcore_barrier()

    for pstart, plen in PHASES:
        pltpu.sync_copy(src_hbm.at[wid, pl.ds(pstart, plen)],
                        src_v.at[pl.ds(0, plen)])
        pltpu.sync_copy(dst_hbm.at[wid, pl.ds(pstart, plen)],
                        dst_v.at[pl.ds(0, plen)])

        for b in range(2):
            pltpu.async_copy(y_hbm.at[src_v.at[b]], rows_v.at[b], gy.at[b])

        def pair(i, carry):
            for b in range(2):
                j = 2 * i + b
                pltpu.async_copy(r_hbm.at[dst_v.at[j]], rvals_v.at[j], gt)
                pltpu.make_async_copy(y_hbm.at[src_v.at[j]], rows_v.at[b],
                                      gy.at[b]).wait()
                pltpu.async_copy(rows_v.at[b], acc_sh.at[dst_v.at[j]],
                                 sy.at[b], add=True)

                @pl.when(j >= 2)
                def _():
                    pltpu.make_async_copy(r_hbm.at[dst_v.at[j - 2]],
                                          rvals_v.at[j - 2], gt).wait()
                    pltpu.async_copy(rvals_v.at[j - 2],
                                     t_sh.at[src_v.at[j - 2]], st, add=True)

                pltpu.make_async_copy(rows_v.at[b], acc_sh.at[dst_v.at[j]],
                                      sy.at[b]).wait()

                @pl.when(j + 2 < plen)
                def _():
                    pltpu.async_copy(y_hbm.at[src_v.at[j + 2]], rows_v.at[b],
                                     gy.at[b])
            return carry

        lax.fori_loop(0, plen // 2, pair, 0)

        for j in (plen - 2, plen - 1):
            pltpu.make_async_copy(r_hbm.at[dst_v.at[j]], rvals_v.at[j],
                                  gt).wait()
            pltpu.async_copy(rvals_v.at[j], t_sh.at[src_v.at[j]], st,
                             add=True)

        def tsw(j, carry):
            pltpu.make_async_copy(rvals_v.at[j], t_sh.at[src_v.at[j]],
                                  st).wait()
            return carry

        lax.fori_loop(0, plen, tsw, 0)

    plsc.subcore_barrier()

    def cpout(i, carry):
        pltpu.sync_copy(acc_sh.at[pl.ds(s * RPT + i * K, K)], rows_v.at[0])
        pltpu.sync_copy(rows_v.at[0], acc_hbm.at[c, pl.ds(s * RPT + i * K, K)])
        return carry

    lax.fori_loop(0, RPT // K, cpout, 0)
    pltpu.sync_copy(acc_sh.at[pl.ds(s * RPT + (RPT // K) * K, RPT % K)],
                    rows_v.at[0, pl.ds(0, RPT % K)])
    pltpu.sync_copy(rows_v.at[0, pl.ds(0, RPT % K)],
                    acc_hbm.at[c, pl.ds(s * RPT + (RPT // K) * K, RPT % K)])

    def tout(i, carry):
        pltpu.sync_copy(t_sh.at[pl.ds(s * RPT + i * K, K)], rvals_v.at[0])
        pltpu.sync_copy(rvals_v.at[0],
                        t_hbm.at[pl.ds(c * N2 + s * RPT + i * K, K)])
        return carry

    lax.fori_loop(0, RPT // K, tout, 0)
    pltpu.sync_copy(t_sh.at[pl.ds(s * RPT + (RPT // K) * K, RPT % K)],
                    rvals_v.at[0, pl.ds(0, RPT % K)])
    pltpu.sync_copy(rvals_v.at[0, pl.ds(0, RPT % K)],
                    t_hbm.at[pl.ds(c * N2 + s * RPT + (RPT // K) * K,
                                   RPT % K)])


def _scale_body(xp_ref, w1_ref, degp_ref, y_ref, r_ref):
    deg = jnp.sum(degp_ref[...], axis=1, keepdims=True) + 1.0
    r = lax.rsqrt(deg)
    xw = jnp.dot(xp_ref[...], w1_ref[...],
                 preferred_element_type=jnp.float32)
    y_ref[...] = r * xw
    r_ref[...] = r


def _final_body(acc_ref, y_ref, tp_ref, r_ref, b1_ref, w2_ref, b2_ref, o_ref):
    r = r_ref[...]
    a = acc_ref[0] + acc_ref[1] + y_ref[...]
    h1 = jnp.maximum(r * a + b1_ref[...], 0.0)
    t = jnp.sum(tp_ref[...], axis=1, keepdims=True) + r
    valid = lax.broadcasted_iota(jnp.int32, (N2, 1), 0) < N
    cvec = jnp.where(valid, r * t, 0.0)
    u = jnp.sum(cvec * h1, axis=0, keepdims=True)
    o_ref[...] = (u * (1.0 / N)) @ w2_ref[...] + b2_ref[...]


def kernel(x, edge_index, W1, b1, W2, b2):
    ei = edge_index.astype(jnp.int32)
    padv = DUMMY + jnp.arange(E_PAD - E, dtype=jnp.int32) % (N2 - N)
    src_flat = jnp.concatenate([ei[0], padv])
    dst_flat = jnp.concatenate([ei[1], padv])
    xp = jnp.zeros((N2, D), jnp.float32).at[:N].set(x)
    ones = jnp.ones((KD,), jnp.float32)
    zcol = jnp.zeros((N2,), jnp.float32)
    zrows = jnp.zeros((N2, D), jnp.float32)

    deg_part = _deg_kernel(dst_flat.reshape(TILES, CHD, KD), ones,
                           zcol).reshape(NC, N2)

    y, r_col = pl.pallas_call(
        _scale_body,
        out_shape=[jax.ShapeDtypeStruct((N2, D), jnp.float32),
                   jax.ShapeDtypeStruct((N2, 1), jnp.float32)],
    )(xp, W1, deg_part.T)

    acc, t_part = _agg_kernel(y, r_col.reshape(N2),
                              src_flat.reshape(TILES, CH, K),
                              dst_flat.reshape(TILES, CH, K), zrows, zcol)

    out = pl.pallas_call(
        _final_body,
        out_shape=jax.ShapeDtypeStruct((1, D), jnp.float32),
    )(acc, y, t_part.reshape(NC, N2).T, r_col, b1.reshape(1, D), W2,
      b2.reshape(1, D))
    return out


# speedup vs baseline: 1.3248x; 1.0259x over previous
import functools

import jax
import jax.numpy as jnp
from jax import lax
from jax.experimental import pallas as pl
from jax.experimental.pallas import tpu as pltpu
from jax.experimental.pallas import tpu_sc as plsc

N = 10000
D = 128
E = 320000
NC = 2
NS = 16
TILES = NC * NS
KD = 128
CHD = 80
E_PAD = TILES * KD * CHD
K = 128
CH = CHD
PHASES = ((0, 40), (40, 40))
PBUF = 40
N2 = 10112
DUMMY = N
RPT = N2 // NS

_mesh = plsc.VectorSubcoreMesh(core_axis_name="c", subcore_axis_name="s",
                               num_cores=NC, num_subcores=NS)


@functools.partial(
    pl.kernel,
    out_type=jax.ShapeDtypeStruct((NC * N2,), jnp.float32),
    mesh=_mesh,
    scratch_types=[
        pltpu.VMEM((CHD, KD), jnp.int32),
        pltpu.VMEM((KD,), jnp.float32),
        pltpu.VMEM((RPT,), jnp.float32),
        pltpu.VMEM_SHARED((N2,), jnp.float32),
        pltpu.SemaphoreType.DMA,
    ],
)
def _deg_kernel(dst_hbm, ones_hbm, zcol_hbm, out_hbm, dst_v, ones_v, zbuf,
                deg_sh, sd):
    c = lax.axis_index("c")
    s = lax.axis_index("s")
    pltpu.sync_copy(dst_hbm.at[c * NS + s], dst_v)
    pltpu.sync_copy(ones_hbm, ones_v)
    pltpu.sync_copy(zcol_hbm.at[pl.ds(s * RPT, RPT)], zbuf)
    pltpu.sync_copy(zbuf, deg_sh.at[pl.ds(s * RPT, RPT)])
    plsc.subcore_barrier()

    def body(j, carry):
        pltpu.async_copy(ones_v, deg_sh.at[dst_v.at[j]], sd, add=True)
        return carry

    lax.fori_loop(0, CHD, body, 0)

    def bodyw(j, carry):
        pltpu.make_async_copy(ones_v, deg_sh.at[dst_v.at[j]], sd).wait()
        return carry

    lax.fori_loop(0, CHD, bodyw, 0)
    plsc.subcore_barrier()
    pltpu.sync_copy(deg_sh.at[pl.ds(s * RPT, RPT)], zbuf)
    pltpu.sync_copy(zbuf, out_hbm.at[pl.ds(c * N2 + s * RPT, RPT)])


@functools.partial(
    pl.kernel,
    out_type=(jax.ShapeDtypeStruct((NC, N2, D), jnp.float32),
              jax.ShapeDtypeStruct((NC * N2,), jnp.float32)),
    mesh=_mesh,
    scratch_types=[
        pltpu.VMEM((PBUF, K), jnp.int32),
        pltpu.VMEM((PBUF, K), jnp.int32),
        pltpu.VMEM((PBUF, K), jnp.float32),
        pltpu.VMEM((2, K, D), jnp.float32),
        pltpu.VMEM_SHARED((N2, D), jnp.float32),
        pltpu.VMEM_SHARED((N2,), jnp.float32),
        pltpu.SemaphoreType.DMA((2,)),
        pltpu.SemaphoreType.DMA((2,)),
        pltpu.SemaphoreType.DMA,
        pltpu.SemaphoreType.DMA,
    ],
)
def _agg_kernel(y_hbm, r_hbm, src_hbm, dst_hbm, zrows_hbm, zcol_hbm,
                acc_hbm, t_hbm,
                src_v, dst_v, rvals_v, rows_v, acc_sh, t_sh, gy, sy, gt, st):
    c = lax.axis_index("c")
    s = lax.axis_index("s")
    wid = c * NS + s

    pltpu.sync_copy(zrows_hbm.at[pl.ds(s * RPT, K)], rows_v.at[0])

    def zb(i, carry):
        pltpu.sync_copy(rows_v.at[0], acc_sh.at[pl.ds(s * RPT + i * K, K)])
        return carry

    lax.fori_loop(0, RPT // K, zb, 0)
    pltpu.sync_copy(rows_v.at[0, pl.ds(0, RPT % K)],
                    acc_sh.at[pl.ds(s * RPT + (RPT // K) * K, RPT % K)])
    pltpu.sync_copy(zcol_hbm.at[pl.ds(s * RPT, K)], rvals_v.at[0])

    def zt(i, carry):
        pltpu.sync_copy(rvals_v.at[0], t_sh.at[pl.ds(s * RPT + i * K, K)])
        return carry

    lax.fori_loop(0, RPT // K, zt, 0)
    pltpu.sync_copy(rvals_v.at[0, pl.ds(0, RPT % K)],
                    t_sh.at[pl.ds(s * RPT + (RPT // K) * K, RPT % K)])
    plsc.subcore_barrier()

    for pstart, plen in PHASES:
        pltpu.sync_copy(src_hbm.at[wid, pl.ds(pstart, plen)],
                        src_v.at[pl.ds(0, plen)])
        pltpu.sync_copy(dst_hbm.at[wid, pl.ds(pstart, plen)],
                        dst_v.at[pl.ds(0, plen)])

        for b in range(2):
            pltpu.async_copy(y_hbm.at[src_v.at[b]], rows_v.at[b], gy.at[b])

        def pair(i, carry):
            for b in range(2):
                j = 2 * i + b
                pltpu.async_copy(r_hbm.at[dst_v.at[j]], rvals_v.at[j], gt)
                pltpu.make_async_copy(y_hbm.at[src_v.at[j]], rows_v.at[b],
                                      gy.at[b]).wait()
                pltpu.async_copy(rows_v.at[b], acc_sh.at[dst_v.at[j]],
                                 sy.at[b], add=True)

                @pl.when(j >= 2)
                def _():
                    pltpu.make_async_copy(r_hbm.at[dst_v.at[j - 2]],
                                          rvals_v.at[j - 2], gt).wait()
                    pltpu.async_copy(rvals_v.at[j - 2],
                                     t_sh.at[src_v.at[j - 2]], st, add=True)

                pltpu.make_async_copy(rows_v.at[b], acc_sh.at[dst_v.at[j]],
                                      sy.at[b]).wait()

                @pl.when(j + 2 < plen)
                def _():
                    pltpu.async_copy(y_hbm.at[src_v.at[j + 2]], rows_v.at[b],
                                     gy.at[b])
            return carry

        lax.fori_loop(0, plen // 2, pair, 0)

        for j in (plen - 2, plen - 1):
            pltpu.make_async_copy(r_hbm.at[dst_v.at[j]], rvals_v.at[j],
                                  gt).wait()
            pltpu.async_copy(rvals_v.at[j], t_sh.at[src_v.at[j]], st,
                             add=True)

        def tsw(j, carry):
            pltpu.make_async_copy(rvals_v.at[j], t_sh.at[src_v.at[j]],
                                  st).wait()
            return carry

        lax.fori_loop(0, plen, tsw, 0)

    plsc.subcore_barrier()

    def cpout(i, carry):
        pltpu.sync_copy(acc_sh.at[pl.ds(s * RPT + i * K, K)], rows_v.at[0])
        pltpu.sync_copy(rows_v.at[0], acc_hbm.at[c, pl.ds(s * RPT + i * K, K)])
        return carry

    lax.fori_loop(0, RPT // K, cpout, 0)
    pltpu.sync_copy(acc_sh.at[pl.ds(s * RPT + (RPT // K) * K, RPT % K)],
                    rows_v.at[0, pl.ds(0, RPT % K)])
    pltpu.sync_copy(rows_v.at[0, pl.ds(0, RPT % K)],
                    acc_hbm.at[c, pl.ds(s * RPT + (RPT // K) * K, RPT % K)])

    def tout(i, carry):
        pltpu.sync_copy(t_sh.at[pl.ds(s * RPT + i * K, K)], rvals_v.at[0])
        pltpu.sync_copy(rvals_v.at[0],
                        t_hbm.at[pl.ds(c * N2 + s * RPT + i * K, K)])
        return carry

    lax.fori_loop(0, RPT // K, tout, 0)
    pltpu.sync_copy(t_sh.at[pl.ds(s * RPT + (RPT // K) * K, RPT % K)],
                    rvals_v.at[0, pl.ds(0, RPT % K)])
    pltpu.sync_copy(rvals_v.at[0, pl.ds(0, RPT % K)],
                    t_hbm.at[pl.ds(c * N2 + s * RPT + (RPT // K) * K,
                                   RPT % K)])


def _scale_body(xp_ref, w1_ref, degp_ref, y_ref, r_ref):
    deg = jnp.sum(degp_ref[...], axis=1, keepdims=True) + 1.0
    r = lax.rsqrt(deg)
    xw = jnp.dot(xp_ref[...], w1_ref[...],
                 preferred_element_type=jnp.float32)
    y_ref[...] = r * xw
    r_ref[...] = r


def _final_body(acc_ref, y_ref, tp_ref, r_ref, b1_ref, w2_ref, b2_ref, o_ref):
    r = r_ref[...]
    a = acc_ref[0] + acc_ref[1] + y_ref[...]
    h1 = jnp.maximum(r * a + b1_ref[...], 0.0)
    t = jnp.sum(tp_ref[...], axis=1, keepdims=True) + r
    valid = lax.broadcasted_iota(jnp.int32, (N2, 1), 0) < N
    cvec = jnp.where(valid, r * t, 0.0)
    u = jnp.sum(cvec * h1, axis=0, keepdims=True)
    o_ref[...] = (u * (1.0 / N)) @ w2_ref[...] + b2_ref[...]


def kernel(x, edge_index, W1, b1, W2, b2):
    ei = edge_index.astype(jnp.int32)
    padv = DUMMY + jnp.arange(E_PAD - E, dtype=jnp.int32) % (N2 - N)
    src_flat = jnp.concatenate([ei[0], padv])
    dst_flat = jnp.concatenate([ei[1], padv])
    xp = jnp.zeros((N2, D), jnp.float32).at[:N].set(x)
    ones = jnp.ones((KD,), jnp.float32)
    zcol = jnp.zeros((N2,), jnp.float32)
    zrows = jnp.zeros((N2, D), jnp.float32)

    deg_part = _deg_kernel(dst_flat.reshape(TILES, CHD, KD), ones,
                           zcol).reshape(NC, N2)

    y, r_col = pl.pallas_call(
        _scale_body,
        out_shape=[jax.ShapeDtypeStruct((N2, D), jnp.float32),
                   jax.ShapeDtypeStruct((N2, 1), jnp.float32)],
    )(xp, W1, deg_part.T)

    acc, t_part = _agg_kernel(y, r_col.reshape(N2),
                              src_flat.reshape(TILES, CH, K),
                              dst_flat.reshape(TILES, CH, K), zrows, zcol)

    out = pl.pallas_call(
        _final_body,
        out_shape=jax.ShapeDtypeStruct((1, D), jnp.float32),
    )(acc, y, t_part.reshape(NC, N2).T, r_col, b1.reshape(1, D), W2,
      b2.reshape(1, D))
    return out
